# Initial kernel scaffold; baseline (speedup 1.0000x reference)
import jax
import jax.numpy as jnp
from jax.experimental import pallas as pl


def kernel(gene_ind_vec, edge_index, gene_embedding, W1, b1, W2, b2):
    raise NotImplementedError("write your pallas kernel here")


# speedup vs baseline: 14.1739x; 14.1739x over previous
import functools

import jax
import jax.numpy as jnp
from jax import lax
from jax.experimental import pallas as pl
from jax.experimental.pallas import tpu as pltpu
from jax.experimental.pallas import tpu_sc as plsc

N_GENES = 10000
D = 128
N_EDGES = 320000

NC = 2
NS = 16
L = 16

NPAD = 10240
EDGES_PER_TILE_DEG = N_EDGES // NS
DEG_CHUNK = 2000
EDGES_PER_TILE_AGG = N_EDGES // (NC * NS)
AGG_CHUNK = 80
AGG_NCHUNK = EDGES_PER_TILE_AGG // AGG_CHUNK
ROWS_PER_TILE = NPAD // NS


def _newton_rsqrt(x):
    i = plsc.bitcast(x, jnp.int32)
    y = plsc.bitcast(jnp.int32(0x5F3759DF) - (i >> 1), jnp.float32)
    for _ in range(3):
        y = y * (1.5 - 0.5 * x * y * y)
    return y


def _deg_kernel(dst_hbm, dinv_hbm, dstbuf, deg_tile, tmp, acc, deg_sh):
    cid = lax.axis_index("c")
    sid = lax.axis_index("s")

    @pl.when(cid == 0)
    def _():
        zeros16 = jnp.zeros((L,), jnp.float32)

        def zloop(i, _):
            deg_tile[pl.ds(i * L, L)] = zeros16
            return 0
        lax.fori_loop(0, NPAD // L, zloop, 0)

        ones = zeros16 + 1.0

        def chunk(j, _):
            pltpu.sync_copy(dst_hbm.at[pl.ds(sid * EDGES_PER_TILE_DEG
                                             + j * DEG_CHUNK, DEG_CHUNK)],
                            dstbuf)

            def scat(k, _):
                idx = dstbuf[pl.ds(k * L, L)]
                plsc.addupdate_scatter(deg_tile, [idx], ones)
                return 0
            lax.fori_loop(0, DEG_CHUNK // L, scat, 0)
            return 0
        lax.fori_loop(0, EDGES_PER_TILE_DEG // DEG_CHUNK, chunk, 0)

        pltpu.sync_copy(deg_tile, deg_sh.at[sid])
        plsc.subcore_barrier()

        stripe = NPAD // NS
        def zacc(i, _):
            acc[pl.ds(i * L, L)] = zeros16
            return 0
        lax.fori_loop(0, stripe // L, zacc, 0)

        for t in range(NS):
            pltpu.sync_copy(deg_sh.at[t, pl.ds(sid * stripe, stripe)], tmp)

            def addl(i, _):
                acc[pl.ds(i * L, L)] = acc[pl.ds(i * L, L)] + tmp[pl.ds(i * L, L)]
                return 0
            lax.fori_loop(0, stripe // L, addl, 0)

        def fin(i, _):
            d = acc[pl.ds(i * L, L)] + 1.0
            acc[pl.ds(i * L, L)] = _newton_rsqrt(d)
            return 0
        lax.fori_loop(0, stripe // L, fin, 0)

        pltpu.sync_copy(acc, dinv_hbm.at[pl.ds(sid * stripe, stripe)])


def _deg_call(dst):
    mesh = plsc.VectorSubcoreMesh(core_axis_name="c", subcore_axis_name="s")

    @functools.partial(
        pl.kernel,
        out_type=jax.ShapeDtypeStruct((NPAD,), jnp.float32),
        mesh=mesh,
        scratch_types=[
            pltpu.VMEM((DEG_CHUNK,), jnp.int32),
            pltpu.VMEM((NPAD,), jnp.float32),
            pltpu.VMEM((NPAD // NS,), jnp.float32),
            pltpu.VMEM((NPAD // NS,), jnp.float32),
            pltpu.VMEM_SHARED((NS, NPAD), jnp.float32),
        ],
        compiler_params=pltpu.CompilerParams(needs_layout_passes=False),
    )
    def call(dst_hbm, dinv_hbm, dstbuf, deg_tile, tmp, acc, deg_sh):
        _deg_kernel(dst_hbm, dinv_hbm, dstbuf, deg_tile, tmp, acc, deg_sh)

    return call(dst)


def _agg_call(hs, src, dst):
    mesh = plsc.VectorSubcoreMesh(core_axis_name="c", subcore_axis_name="s")

    @functools.partial(
        pl.kernel,
        out_type=jax.ShapeDtypeStruct((NC, NPAD, D), jnp.float32),
        mesh=mesh,
        scratch_types=[
            pltpu.VMEM((AGG_CHUNK,), jnp.int32),
            pltpu.VMEM((AGG_CHUNK,), jnp.int32),
            pltpu.VMEM((AGG_CHUNK, D), jnp.float32),
            pltpu.VMEM((128, D), jnp.float32),
            pltpu.VMEM_SHARED((NPAD, D), jnp.float32),
            pltpu.SemaphoreType.DMA,
        ],
        compiler_params=pltpu.CompilerParams(needs_layout_passes=False),
    )
    def call(hs_hbm, src_hbm, dst_hbm, out_hbm, sidx, didx, rows, zbuf, agg_sh,
             sem):
        cid = lax.axis_index("c")
        sid = lax.axis_index("s")

        def zl(i, _):
            def zc(j, _):
                zbuf[i, pl.ds(j * L, L)] = jnp.zeros((L,), jnp.float32)
                return 0
            lax.fori_loop(0, D // L, zc, 0)
            return 0
        lax.fori_loop(0, 128, zl, 0)
        for r in range(5):
            pltpu.sync_copy(zbuf, agg_sh.at[pl.ds(sid * ROWS_PER_TILE + r * 128,
                                                  128)])
        plsc.subcore_barrier()

        gbase = (cid * NS + sid) * EDGES_PER_TILE_AGG

        def chunk(k, _):
            off = gbase + k * AGG_CHUNK
            pltpu.sync_copy(src_hbm.at[pl.ds(off, AGG_CHUNK)], sidx)
            pltpu.async_copy(hs_hbm.at[sidx], rows, sem).wait()
            pltpu.sync_copy(dst_hbm.at[pl.ds(off, AGG_CHUNK)], didx)
            pltpu.sync_copy(rows, agg_sh.at[didx], add=True)
            return 0
        lax.fori_loop(0, AGG_NCHUNK, chunk, 0)

        plsc.subcore_barrier()
        pltpu.sync_copy(agg_sh.at[pl.ds(sid * ROWS_PER_TILE, ROWS_PER_TILE)],
                        out_hbm.at[cid, pl.ds(sid * ROWS_PER_TILE,
                                              ROWS_PER_TILE)])

    return call(hs, src, dst)


_BLK = 1000
_GRID = N_GENES // _BLK


def _tc1_body(x_ref, dinv_ref, w_ref, o_ref):
    o_ref[...] = jnp.dot(x_ref[...] * dinv_ref[...], w_ref[...],
                         preferred_element_type=jnp.float32)


def _tc2_body(p0_ref, p1_ref, hs_ref, dinv_ref, b_ref, w_ref, o_ref):
    agg = (p0_ref[...] + p1_ref[...] + hs_ref[...]) * dinv_ref[...]
    x1 = jnp.maximum(agg + b_ref[...], 0.0)
    o_ref[...] = jnp.dot(x1 * dinv_ref[...], w_ref[...],
                         preferred_element_type=jnp.float32)


def _tc3_body(p0_ref, p1_ref, hs_ref, dinv_ref, b_ref, o_ref):
    o_ref[...] = ((p0_ref[...] + p1_ref[...] + hs_ref[...]) * dinv_ref[...]
                  + b_ref[...])


def _row_spec():
    return pl.BlockSpec((_BLK, D), lambda i: (i, 0))


def _full_spec():
    return pl.BlockSpec((D, D), lambda i: (0, 0))


def _bias_spec():
    return pl.BlockSpec((1, D), lambda i: (0, 0))


def _tc1(x, dinv_bc, w):
    return pl.pallas_call(
        _tc1_body,
        grid=(_GRID,),
        in_specs=[_row_spec(), _row_spec(), _full_spec()],
        out_specs=_row_spec(),
        out_shape=jax.ShapeDtypeStruct((N_GENES, D), jnp.float32),
    )(x, dinv_bc, w)


def _tc2(p0, p1, hs, dinv_bc, b, w):
    return pl.pallas_call(
        _tc2_body,
        grid=(_GRID,),
        in_specs=[_row_spec(), _row_spec(), _row_spec(), _row_spec(),
                  _bias_spec(), _full_spec()],
        out_specs=_row_spec(),
        out_shape=jax.ShapeDtypeStruct((N_GENES, D), jnp.float32),
    )(p0, p1, hs, dinv_bc, b, w)


def _tc3(p0, p1, hs, dinv_bc, b):
    return pl.pallas_call(
        _tc3_body,
        grid=(_GRID,),
        in_specs=[_row_spec(), _row_spec(), _row_spec(), _row_spec(),
                  _bias_spec()],
        out_specs=_row_spec(),
        out_shape=jax.ShapeDtypeStruct((N_GENES, D), jnp.float32),
    )(p0, p1, hs, dinv_bc, b)


def kernel(gene_ind_vec, edge_index, gene_embedding, W1, b1, W2, b2):
    src = edge_index[0]
    dst = edge_index[1]

    dinv_pad = _deg_call(dst)
    dinv_bc = jnp.broadcast_to(dinv_pad[:N_GENES, None], (N_GENES, D))

    hs1 = _tc1(gene_embedding, dinv_bc, W1)
    agg1 = _agg_call(hs1, src, dst)
    hs2 = _tc2(agg1[0], agg1[1], hs1, dinv_bc, b1.reshape(1, D), W2)
    agg2 = _agg_call(hs2, src, dst)
    out = _tc3(agg2[0], agg2[1], hs2, dinv_bc, b2.reshape(1, D))
    return out
